# baseline (speedup 1.0000x reference)
import jax
import jax.numpy as jnp
from jax.experimental import pallas as pl
from jax.experimental.pallas import tpu as pltpu

_BLOCK = 2048


def _copy_block(in_ref, out_hbm, sem):
    i = pl.program_id(0)
    copy = pltpu.make_async_copy(
        in_ref, out_hbm.at[pl.ds(i * _BLOCK, _BLOCK)], sem
    )
    copy.start()
    copy.wait()


def kernel(x, embedding):
    seq_len = x.shape[1]
    features = embedding.shape[1]
    return pl.pallas_call(
        _copy_block,
        grid=(seq_len // _BLOCK,),
        in_specs=[pl.BlockSpec((_BLOCK, features), lambda i: (i, 0))],
        out_specs=pl.BlockSpec(memory_space=pl.ANY),
        scratch_shapes=[pltpu.SemaphoreType.DMA],
        out_shape=jax.ShapeDtypeStruct((seq_len, features), embedding.dtype),
    )(embedding)
